# baseline (speedup 1.0000x reference)
import functools

import jax
import jax.numpy as jnp
from jax import lax
from jax.experimental import pallas as pl
from jax.experimental.pallas import tpu as pltpu
from jax.experimental.pallas import tpu_sc as plsc

N_USER = 25000
N_ITEM = 25000
N_NODES = N_USER + N_ITEM
N_EDGES = 800000
VT_DIM = 128
CAT_EMBED_DIM = 32
HIDDEN = 64

NC = 2
NS = 16
LANES = 16

GB = 8
PAIRS = 25
BLOCKS_PT = 2 * PAIRS
ROWS_PT = GB * BLOCKS_PT
E_ROWS = NS * ROWS_PT
E_PAD = E_ROWS * 128
E_ROWS_AL = E_ROWS + 2 * GB

FH = HIDDEN // NC
ACC2 = 50176
ZPT2 = ACC2 // NS
N_GARB = ACC2 - N_NODES

DROWS_PT = ROWS_PT // 2
DBLOCKS = DROWS_PT // GB

RING = 4

CAT_ROWS_PW = 8
CAT_PW = CAT_ROWS_PW * 128
CAT_PAD = NC * NS * CAT_PW

_MESH = plsc.VectorSubcoreMesh(core_axis_name="c", subcore_axis_name="s")
_SC_PARAMS = pltpu.CompilerParams(use_tc_tiling_on_sc=False)

_CHUNKS_3136_32 = tuple((i * 384, 384) for i in range(8)) + ((3072, 64),)
_CHUNKS_3136_16 = tuple((i * 512, 512) for i in range(6)) + ((3072, 64),)


@functools.partial(
    pl.kernel,
    out_type=jax.ShapeDtypeStruct((NC * ACC2, FH), jnp.float32),
    mesh=_MESH,
    compiler_params=_SC_PARAMS,
    scratch_types=[
        pltpu.VMEM((2 * GB, 128), jnp.int32),
        pltpu.VMEM((2 * GB, 128), jnp.int32),
        pltpu.VMEM((2 * GB, 128), jnp.int32),
        pltpu.VMEM((2 * GB, 128), jnp.int32),
        pltpu.VMEM((RING * 128, FH), jnp.float32),
        pltpu.VMEM_SHARED((ACC2, FH), jnp.float32),
        pltpu.SemaphoreType.DMA,
        pltpu.SemaphoreType.DMA,
        pltpu.SemaphoreType.DMA,
    ],
)
def _sc_spmv(xs_hbm, src_hbm, dst_hbm, z_hbm, out_hbm,
             sidx, dstb, gidx, lidx, ring, acc, gsem, isem, ssem):
    cid = lax.axis_index("c")
    sid = lax.axis_index("s")
    tile0 = sid * ROWS_PT

    def idx_copies(pair):
        row0 = tile0 + pair * (2 * GB)
        return ((src_hbm.at[pl.ds(row0, 2 * GB)], sidx),
                (dst_hbm.at[pl.ds(row0, 2 * GB)], dstb))

    def load_idx(pair):
        for s, d in idx_copies(pair):
            pltpu.async_copy(s, d, isem)

    def wait_idx(pair):
        for s, d in idx_copies(pair):
            pltpu.make_async_copy(s, d, isem).wait()

    load_idx(0)
    zcps = [pltpu.async_copy(
        z_hbm.at[pl.ds(0, ln)], acc.at[pl.ds(sid * ZPT2 + off, ln)], gsem)
        for off, ln in _CHUNKS_3136_32]
    for c in zcps:
        c.wait()
    plsc.subcore_barrier()

    base = cid * N_NODES

    def remap(h):
        for j in range(GB):
            r = h * GB + j
            for m in range(128 // LANES):
                sl = pl.ds(m * LANES, LANES)
                lidx[r, sl] = dstb[r, sl]
                gidx[r, sl] = sidx[r, sl] + base

    def gather(s):
        return pltpu.async_copy(xs_hbm.at[gidx.at[s]],
                                ring.at[pl.ds((s % RING) * 128, 128)], gsem)

    def scat(s):
        return pltpu.async_copy(ring.at[pl.ds((s % RING) * 128, 128)],
                                acc.at[lidx.at[s]], ssem, add=True)

    def body(pair, carry):
        wait_idx(pair)
        remap(0)
        remap(1)
        load_idx(pair + 1)
        cps = [None] * (2 * GB)
        scs = [None] * (2 * GB)
        for s in range(2 * GB):
            if s >= RING:
                scs[s - RING].wait()
            cps[s] = gather(s)
            if s >= 1:
                cps[s - 1].wait()
                scs[s - 1] = scat(s - 1)
        cps[2 * GB - 1].wait()
        scs[2 * GB - 1] = scat(2 * GB - 1)
        for s in range(2 * GB - RING, 2 * GB):
            scs[s].wait()
        return carry

    lax.fori_loop(0, PAIRS, body, 0)
    wait_idx(PAIRS)
    plsc.subcore_barrier()

    pltpu.sync_copy(acc.at[pl.ds(sid * ZPT2, ZPT2)],
                    out_hbm.at[pl.ds(cid * ACC2 + sid * ZPT2, ZPT2)])


@functools.partial(
    pl.kernel,
    out_type=(jax.ShapeDtypeStruct((CAT_PAD, CAT_EMBED_DIM), jnp.float32),
              jax.ShapeDtypeStruct((NC * ACC2, LANES), jnp.float32)),
    mesh=_MESH,
    compiler_params=_SC_PARAMS,
    scratch_types=[
        pltpu.VMEM((CAT_ROWS_PW, 128), jnp.int32),
        pltpu.VMEM((CAT_PW, CAT_EMBED_DIM), jnp.float32),
        pltpu.VMEM((2 * GB, 128), jnp.int32),
        pltpu.VMEM((128, LANES), jnp.float32),
        pltpu.VMEM((512, LANES), jnp.float32),
        pltpu.VMEM_SHARED((ACC2, LANES), jnp.float32),
        pltpu.SemaphoreType.DMA,
        pltpu.SemaphoreType.DMA,
        pltpu.SemaphoreType.DMA,
    ],
)
def _sc_pre(table_hbm, cidx_hbm, dst_hbm, ones_hbm, z_hbm,
            cat_out, deg_out, csidx, crows, dstb, ones_v, buf, dacc,
            csem, isem, ssem):
    cid = lax.axis_index("c")
    sid = lax.axis_index("s")
    wid = sid * NC + cid
    row0 = (cid * NS + sid) * DROWS_PT

    pltpu.sync_copy(cidx_hbm.at[pl.ds(wid * CAT_ROWS_PW, CAT_ROWS_PW)], csidx)
    cat_cps = [
        pltpu.async_copy(table_hbm.at[csidx.at[j]],
                         crows.at[pl.ds(j * 128, 128)], csem)
        for j in range(CAT_ROWS_PW)
    ]

    def didx_copy(b):
        return (dst_hbm.at[pl.ds(row0 + b * GB, GB)],
                dstb.at[pl.ds((b % 2) * GB, GB)])

    s, d = didx_copy(0)
    pltpu.async_copy(s, d, isem)

    pltpu.sync_copy(ones_hbm, ones_v)
    pltpu.sync_copy(z_hbm, buf)
    for off, ln in _CHUNKS_3136_16:
        pltpu.sync_copy(buf.at[pl.ds(0, ln)],
                        dacc.at[pl.ds(sid * ZPT2 + off, ln)])
    plsc.subcore_barrier()

    prev = []
    for b in range(DBLOCKS):
        s, d = didx_copy(b)
        pltpu.make_async_copy(s, d, isem).wait()
        scs = [pltpu.async_copy(ones_v, dacc.at[dstb.at[(b % 2) * GB + r]],
                                ssem, add=True)
               for r in range(GB)]
        for c in prev:
            c.wait()
        if b + 1 < DBLOCKS:
            s, d = didx_copy(b + 1)
            pltpu.async_copy(s, d, isem)
        prev = scs
    for c in prev:
        c.wait()
    plsc.subcore_barrier()

    for off, ln in _CHUNKS_3136_16:
        pltpu.sync_copy(dacc.at[pl.ds(sid * ZPT2 + off, ln)],
                        buf.at[pl.ds(0, ln)])
        pltpu.sync_copy(buf.at[pl.ds(0, ln)],
                        deg_out.at[pl.ds(cid * ACC2 + sid * ZPT2 + off, ln)])

    for cp in cat_cps:
        cp.wait()
    pltpu.sync_copy(crows, cat_out.at[pl.ds(wid * CAT_PW, CAT_PW)])


_FB = 1000
_UB = N_USER // _FB


def _x0_body(user_ref, vt_ref, cat_ref, wv_ref, wc_ref, o_ref):
    dn = (((1,), (1,)), ((), ()))

    @pl.when(pl.program_id(0) < _UB)
    def _():
        o_ref[...] = user_ref[...]

    @pl.when(pl.program_id(0) >= _UB)
    def _():
        o_ref[...] = (
            lax.dot_general(vt_ref[...], wv_ref[...], dn,
                            preferred_element_type=jnp.float32)
            + lax.dot_general(cat_ref[...], wc_ref[...], dn,
                              preferred_element_type=jnp.float32)
        )


def _tc_x0(user, vt, cat_emb, wv, wc):
    return pl.pallas_call(
        _x0_body,
        grid=(N_NODES // _FB,),
        in_specs=[
            pl.BlockSpec((_FB, HIDDEN), lambda i: (jnp.minimum(i, _UB - 1), 0)),
            pl.BlockSpec((_FB, VT_DIM),
                         lambda i: (jnp.maximum(i - _UB, 0), 0)),
            pl.BlockSpec((_FB, CAT_EMBED_DIM),
                         lambda i: (jnp.maximum(i - _UB, 0), 0)),
            pl.BlockSpec((HIDDEN, VT_DIM), lambda i: (0, 0)),
            pl.BlockSpec((HIDDEN, CAT_EMBED_DIM), lambda i: (0, 0)),
        ],
        out_specs=pl.BlockSpec((_FB, HIDDEN), lambda i: (i, 0)),
        out_shape=jax.ShapeDtypeStruct((N_NODES, HIDDEN), jnp.float32),
    )(user, vt, cat_emb, wv, wc)


def _combine_body(leaky, alo_ref, ahi_ref, deg_ref, x_ref, wll_ref, wlh_ref,
                  wr_ref, b_ref, o_ref):
    dn = (((1,), (1,)), ((), ()))
    inv = 1.0 / jnp.maximum(deg_ref[:, 0:1], 1.0)
    y = (
        lax.dot_general(alo_ref[...] * inv, wll_ref[...], dn,
                        preferred_element_type=jnp.float32)
        + lax.dot_general(ahi_ref[...] * inv, wlh_ref[...], dn,
                          preferred_element_type=jnp.float32)
        + lax.dot_general(x_ref[...], wr_ref[...], dn,
                          preferred_element_type=jnp.float32)
        + b_ref[...]
    )
    if leaky:
        y = jnp.where(y >= 0.0, y, 0.01 * y)
    o_ref[...] = y


def _tc_combine(alo, ahi, deg, x, wl, wr, b, leaky):
    return pl.pallas_call(
        functools.partial(_combine_body, leaky),
        grid=(N_NODES // _FB,),
        in_specs=[
            pl.BlockSpec((_FB, FH), lambda i: (i, 0)),
            pl.BlockSpec((_FB, FH), lambda i: (i, 0)),
            pl.BlockSpec((_FB, LANES), lambda i: (i, 0)),
            pl.BlockSpec((_FB, HIDDEN), lambda i: (i, 0)),
            pl.BlockSpec((HIDDEN, FH), lambda i: (0, 0)),
            pl.BlockSpec((HIDDEN, FH), lambda i: (0, 0)),
            pl.BlockSpec((HIDDEN, HIDDEN), lambda i: (0, 0)),
            pl.BlockSpec((1, HIDDEN), lambda i: (0, 0)),
        ],
        out_specs=pl.BlockSpec((_FB, HIDDEN), lambda i: (i, 0)),
        out_shape=jax.ShapeDtypeStruct((N_NODES, HIDDEN), jnp.float32),
    )(alo, ahi, deg, x, wl[:, :FH], wl[:, FH:], wr, b)


def _halves(a):
    return a[:N_NODES], a[ACC2:ACC2 + N_NODES]


def kernel(vt_feature, cat_indices, cat_offsets, edge_index, cat_table,
           fuse_W, user, conv1_Wl, conv1_Wr, conv1_b, conv2_Wl, conv2_Wr,
           conv2_b):
    del cat_offsets

    src = edge_index[0].astype(jnp.int32)
    dst = edge_index[1].astype(jnp.int32)
    pad_n = E_PAD - N_EDGES
    pad_src = jnp.arange(pad_n, dtype=jnp.int32) % N_NODES
    pad_dst = N_NODES + (jnp.arange(pad_n, dtype=jnp.int32) % N_GARB)
    src2d = jnp.pad(
        jnp.concatenate([src, pad_src]).reshape(E_ROWS, 128),
        ((0, E_ROWS_AL - E_ROWS), (0, 0)))
    dst2d = jnp.pad(
        jnp.concatenate([dst, pad_dst]).reshape(E_ROWS, 128),
        ((0, E_ROWS_AL - E_ROWS), (0, 0)), constant_values=N_NODES)
    cat2d = jnp.pad(cat_indices.astype(jnp.int32),
                    (0, CAT_PAD - N_ITEM)).reshape(CAT_PAD // 128, 128)

    z32 = jnp.zeros((384, FH), jnp.float32)
    z16 = jnp.zeros((512, LANES), jnp.float32)
    ones16 = jnp.ones((128, LANES), jnp.float32)

    cat_emb, deg2 = _sc_pre(cat_table, cat2d, dst2d, ones16, z16)
    dlo, dhi = _halves(deg2)
    deg = dlo + dhi

    wv = fuse_W[:, :VT_DIM]
    wc = fuse_W[:, VT_DIM:]
    x0 = _tc_x0(user, vt_feature, cat_emb[:N_ITEM], wv, wc)

    b1 = conv1_b.reshape(1, HIDDEN)
    b2 = conv2_b.reshape(1, HIDDEN)

    x0s = jnp.concatenate([x0[:, :FH], x0[:, FH:]], axis=0)
    a1lo, a1hi = _halves(_sc_spmv(x0s, src2d, dst2d, z32))
    x1 = _tc_combine(a1lo, a1hi, deg, x0, conv1_Wl, conv1_Wr, b1, leaky=True)
    x1s = jnp.concatenate([x1[:, :FH], x1[:, FH:]], axis=0)
    a2lo, a2hi = _halves(_sc_spmv(x1s, src2d, dst2d, z32))
    x2 = _tc_combine(a2lo, a2hi, deg, x1, conv2_Wl, conv2_Wr, b2, leaky=False)
    return x2
